# baseline (speedup 1.0000x reference)
import functools

import jax
import jax.numpy as jnp
from jax import lax
from jax.experimental import pallas as pl
from jax.experimental.pallas import tpu as pltpu
from jax.experimental.pallas import tpu_sc as plsc

B = 4096
L = 200
E = 32
LP = 208
NC = 2
NS = 16
NW = NC * NS
UPW = B // NW
NCHUNK = LP // 16
REG_SCALE = 1e-05


def _sc_gather_dot(user, items, user_embedding, item_embedding):
  mesh = plsc.VectorSubcoreMesh(core_axis_name="c", subcore_axis_name="s")

  @functools.partial(
      pl.kernel,
      out_type=(
          jax.ShapeDtypeStruct((B, LP), jnp.float32),
          jax.ShapeDtypeStruct((NW, 16), jnp.float32),
      ),
      mesh=mesh,
      scratch_types=[
          pltpu.VMEM((UPW,), jnp.int32),
          pltpu.VMEM((UPW, E), jnp.float32),
          pltpu.VMEM((UPW, L), jnp.int32),
          pltpu.VMEM((LP, E), jnp.float32),
          pltpu.VMEM((UPW, LP), jnp.float32),
          pltpu.VMEM((16,), jnp.float32),
          pltpu.SemaphoreType.DMA,
      ],
      compiler_params=pltpu.CompilerParams(
          needs_layout_passes=False, use_tc_tiling_on_sc=False),
  )
  def k(user_hbm, items_hbm, uemb_hbm, iemb_hbm, logits_hbm, ssq_hbm,
        uidx_v, urows_v, iidx_v, rows_v, lout_v, ssq_v, sem):
    wid = lax.axis_index("c") * NS + lax.axis_index("s")
    base = wid * UPW
    pltpu.sync_copy(user_hbm.at[pl.ds(base, UPW)], uidx_v)
    pltpu.sync_copy(items_hbm.at[pl.ds(base, UPW)], iidx_v)
    pltpu.async_copy(uemb_hbm.at[uidx_v], urows_v, sem).wait()

    zero = jnp.zeros((16,), jnp.float32)
    for r in range(L, LP):
      rows_v[r, pl.ds(0, 16)] = zero
      rows_v[r, pl.ds(16, 16)] = zero

    iota16 = lax.iota(jnp.int32, 16)

    def per_user(u, ssq):
      pltpu.async_copy(iemb_hbm.at[iidx_v.at[u]], rows_v.at[pl.ds(0, L)],
                       sem).wait()
      u0 = urows_v[u, pl.ds(0, 16)]
      u1 = urows_v[u, pl.ds(16, 16)]
      ssq = ssq + u0 * u0 + u1 * u1
      us = [u0[e] for e in range(16)] + [u1[e] for e in range(16)]
      for c in range(NCHUNK):
        ridx = iota16 + (c * 16)
        acc = zero
        for e in range(E):
          v = plsc.load_gather(rows_v, [ridx, jnp.full((16,), e, jnp.int32)])
          acc = acc + us[e] * v
          ssq = ssq + v * v
        lout_v[u, pl.ds(c * 16, 16)] = acc
      return ssq

    ssq = lax.fori_loop(0, UPW, per_user, zero)
    ssq_v[...] = ssq
    pltpu.sync_copy(lout_v, logits_hbm.at[pl.ds(base, UPW)])
    pltpu.sync_copy(ssq_v, ssq_hbm.at[wid])

  return k(user, items, user_embedding, item_embedding)


RB = 128
NG = B // RB


def _tc_loss(logits, reward_pad, ssq):
  def body(lg_ref, rw_ref, ssq_ref, gan_ref, reg_ref, acc_ref):
    i = pl.program_id(0)

    @pl.when(i == 0)
    def _():
      acc_ref[0] = 0.0

    lg = lg_ref[...]
    rw = rw_ref[...]
    mask = lax.broadcasted_iota(jnp.int32, (RB, LP), 1) < L
    lgm = jnp.where(mask, lg, -1e30)
    mx = jnp.max(lgm, axis=1, keepdims=True)
    ex = jnp.where(mask, jnp.exp(lgm - mx), 0.0)
    se = jnp.sum(ex, axis=1, keepdims=True)
    lse = mx + jnp.log(se)
    rsum = jnp.sum(rw, axis=1, keepdims=True)
    part = jnp.sum(lg * rw) - jnp.sum(lse * rsum)
    acc_ref[0] = acc_ref[0] + part

    @pl.when(i == NG - 1)
    def _():
      gan_ref[0, 0] = -acc_ref[0] / (B * L)
      reg_ref[0, 0] = REG_SCALE * 0.5 * jnp.sum(ssq_ref[...])

  gan, reg = pl.pallas_call(
      body,
      grid=(NG,),
      in_specs=[
          pl.BlockSpec((RB, LP), lambda i: (i, 0)),
          pl.BlockSpec((RB, LP), lambda i: (i, 0)),
          pl.BlockSpec((NW, 16), lambda i: (0, 0)),
      ],
      out_specs=[
          pl.BlockSpec(memory_space=pltpu.SMEM),
          pl.BlockSpec(memory_space=pltpu.SMEM),
      ],
      out_shape=[
          jax.ShapeDtypeStruct((1, 1), jnp.float32),
          jax.ShapeDtypeStruct((1, 1), jnp.float32),
      ],
      scratch_shapes=[pltpu.SMEM((2,), jnp.float32)],
  )(logits, reward_pad, ssq)
  return gan[0, 0], reg[0, 0]


def kernel(user, items, reward, user_embedding, item_embedding):
  user = user.astype(jnp.int32)
  items = items.astype(jnp.int32)
  reward_pad = jnp.pad(reward.astype(jnp.float32), ((0, 0), (0, LP - L)))
  logits, ssq = _sc_gather_dot(user, items, user_embedding, item_embedding)
  return _tc_loss(logits, reward_pad, ssq)


# speedup vs baseline: 1.2122x; 1.2122x over previous
import functools

import jax
import jax.numpy as jnp
from jax import lax
from jax.experimental import pallas as pl
from jax.experimental.pallas import tpu as pltpu
from jax.experimental.pallas import tpu_sc as plsc

B = 4096
L = 200
E = 32
LP = 208
NC = 2
NS = 16
NW = NC * NS
UPW = B // NW
NCHUNK = LP // 16
EP = 33
REG_SCALE = 1e-05


def _sc_gather_dot(user, items, user_embedding, item_embedding):
  mesh = plsc.VectorSubcoreMesh(core_axis_name="c", subcore_axis_name="s")

  @functools.partial(
      pl.kernel,
      out_type=(
          jax.ShapeDtypeStruct((B, LP), jnp.float32),
          jax.ShapeDtypeStruct((NW, 16), jnp.float32),
      ),
      mesh=mesh,
      scratch_types=[
          pltpu.VMEM((UPW,), jnp.int32),
          pltpu.VMEM((UPW, E), jnp.float32),
          pltpu.VMEM((UPW, L), jnp.int32),
          pltpu.VMEM((LP, E), jnp.float32),
          pltpu.VMEM((LP, E), jnp.float32),
          pltpu.VMEM((UPW, LP), jnp.float32),
          pltpu.VMEM((16,), jnp.float32),
          pltpu.SemaphoreType.DMA,
          pltpu.SemaphoreType.DMA,
          pltpu.SemaphoreType.DMA,
      ],
      compiler_params=pltpu.CompilerParams(
          needs_layout_passes=False, use_tc_tiling_on_sc=False),
  )
  def k(user_hbm, items_hbm, uemb_hbm, iemb_hbm, logits_hbm, ssq_hbm,
        uidx_v, urows_v, iidx_v, rows0, rows1, lout_v, ssq_v,
        sem0, sem1, semu):
    wid = lax.axis_index("c") * NS + lax.axis_index("s")
    base = wid * UPW
    pltpu.sync_copy(items_hbm.at[pl.ds(base, UPW)], iidx_v)
    pltpu.sync_copy(user_hbm.at[pl.ds(base, UPW)], uidx_v)
    pltpu.async_copy(uemb_hbm.at[uidx_v], urows_v, semu)

    zero = jnp.zeros((16,), jnp.float32)
    for rbuf in (rows0, rows1):
      for r in range(L, LP):
        rbuf[r, pl.ds(0, 16)] = zero
        rbuf[r, pl.ds(16, 16)] = zero

    iota16 = lax.iota(jnp.int32, 16)

    def issue(u, rbuf, sem):
      pltpu.async_copy(iemb_hbm.at[iidx_v.at[u]], rbuf.at[pl.ds(0, L)], sem)

    def wait(rbuf, sem):
      pltpu.make_async_copy(iemb_hbm.at[iidx_v.at[0]], rbuf.at[pl.ds(0, L)],
                            sem).wait()

    rots = [(iota16 + e) & (E - 1) for e in range(E)]
    rows = [iota16 + (c * 16) for c in range(NCHUNK)]

    def compute(u, rbuf, ssq):
      u0 = urows_v[u, pl.ds(0, 16)]
      u1 = urows_v[u, pl.ds(16, 16)]
      ssq = ssq + u0 * u0 + u1 * u1
      uu = jnp.broadcast_to(u, (16,)).astype(jnp.int32)
      accs = [zero] * NCHUNK
      sq = [zero, zero, zero, zero]
      for e in range(E):
        urot = plsc.load_gather(urows_v, [uu, rots[e]])
        for c in range(NCHUNK):
          v = plsc.load_gather(rbuf, [rows[c], rots[e]])
          accs[c] = accs[c] + urot * v
          sq[(e + c) % 4] = sq[(e + c) % 4] + v * v
      for c in range(NCHUNK):
        lout_v[u, pl.ds(c * 16, 16)] = accs[c]
      return ssq + ((sq[0] + sq[1]) + (sq[2] + sq[3]))

    issue(0, rows0, sem0)
    pltpu.make_async_copy(uemb_hbm.at[uidx_v], urows_v, semu).wait()

    def pair(i, ssq):
      u0 = 2 * i
      issue(u0 + 1, rows1, sem1)
      wait(rows0, sem0)
      ssq = compute(u0, rows0, ssq)

      @pl.when(i < UPW // 2 - 1)
      def _():
        issue(u0 + 2, rows0, sem0)

      wait(rows1, sem1)
      ssq = compute(u0 + 1, rows1, ssq)
      return ssq

    ssq = lax.fori_loop(0, UPW // 2, pair, zero)
    ssq_v[...] = ssq
    pltpu.sync_copy(lout_v, logits_hbm.at[pl.ds(base, UPW)])
    pltpu.sync_copy(ssq_v, ssq_hbm.at[wid])

  return k(user, items, user_embedding, item_embedding)


RB = 128
NG = B // RB


def _tc_loss(logits, reward_pad, ssq):
  def body(lg_ref, rw_ref, ssq_ref, gan_ref, reg_ref, acc_ref):
    i = pl.program_id(0)

    @pl.when(i == 0)
    def _():
      acc_ref[0] = 0.0

    lg = lg_ref[...]
    rw = rw_ref[...]
    mask = lax.broadcasted_iota(jnp.int32, (RB, LP), 1) < L
    lgm = jnp.where(mask, lg, -1e30)
    mx = jnp.max(lgm, axis=1, keepdims=True)
    ex = jnp.where(mask, jnp.exp(lgm - mx), 0.0)
    se = jnp.sum(ex, axis=1, keepdims=True)
    lse = mx + jnp.log(se)
    rsum = jnp.sum(rw, axis=1, keepdims=True)
    part = jnp.sum(lg * rw) - jnp.sum(lse * rsum)
    acc_ref[0] = acc_ref[0] + part

    @pl.when(i == NG - 1)
    def _():
      gan_ref[0, 0] = -acc_ref[0] / (B * L)
      reg_ref[0, 0] = REG_SCALE * 0.5 * jnp.sum(ssq_ref[...])

  gan, reg = pl.pallas_call(
      body,
      grid=(NG,),
      in_specs=[
          pl.BlockSpec((RB, LP), lambda i: (i, 0)),
          pl.BlockSpec((RB, LP), lambda i: (i, 0)),
          pl.BlockSpec((NW, 16), lambda i: (0, 0)),
      ],
      out_specs=[
          pl.BlockSpec(memory_space=pltpu.SMEM),
          pl.BlockSpec(memory_space=pltpu.SMEM),
      ],
      out_shape=[
          jax.ShapeDtypeStruct((1, 1), jnp.float32),
          jax.ShapeDtypeStruct((1, 1), jnp.float32),
      ],
      scratch_shapes=[pltpu.SMEM((2,), jnp.float32)],
  )(logits, reward_pad, ssq)
  return gan[0, 0], reg[0, 0]


def kernel(user, items, reward, user_embedding, item_embedding):
  user = user.astype(jnp.int32)
  items = items.astype(jnp.int32)
  reward_pad = jnp.pad(reward.astype(jnp.float32), ((0, 0), (0, LP - L)))
  logits, ssq = _sc_gather_dot(user, items, user_embedding, item_embedding)
  return _tc_loss(logits, reward_pad, ssq)


# speedup vs baseline: 1.2536x; 1.0342x over previous
import functools

import jax
import jax.numpy as jnp
from jax import lax
from jax.experimental import pallas as pl
from jax.experimental.pallas import tpu as pltpu
from jax.experimental.pallas import tpu_sc as plsc

B = 4096
L = 200
E = 32
LP = 208
NC = 2
NS = 16
NW = NC * NS
UPW = B // NW
NCHUNK = LP // 16
NROWS = 1000000
REG_SCALE = 1e-05


def _sc_gather_dot(user2a, user2b, items2a, items2b, uemb2, iemb2):
  mesh = plsc.VectorSubcoreMesh(core_axis_name="c", subcore_axis_name="s")

  @functools.partial(
      pl.kernel,
      out_type=(
          jax.ShapeDtypeStruct((B, LP), jnp.float32),
          jax.ShapeDtypeStruct((NW, 16), jnp.float32),
      ),
      mesh=mesh,
      scratch_types=[
          pltpu.VMEM((UPW,), jnp.int32),
          pltpu.VMEM((UPW,), jnp.int32),
          pltpu.VMEM((UPW, 16), jnp.float32),
          pltpu.VMEM((UPW, 16), jnp.float32),
          pltpu.VMEM((UPW, L), jnp.int32),
          pltpu.VMEM((UPW, L), jnp.int32),
          pltpu.VMEM((LP, 16), jnp.float32),
          pltpu.VMEM((LP, 16), jnp.float32),
          pltpu.VMEM((LP, 16), jnp.float32),
          pltpu.VMEM((LP, 16), jnp.float32),
          pltpu.VMEM((UPW, LP), jnp.float32),
          pltpu.VMEM((16,), jnp.float32),
          pltpu.SemaphoreType.DMA,
          pltpu.SemaphoreType.DMA,
          pltpu.SemaphoreType.DMA,
      ],
      compiler_params=pltpu.CompilerParams(
          needs_layout_passes=False, use_tc_tiling_on_sc=False),
  )
  def k(u2a_hbm, u2b_hbm, it2a_hbm, it2b_hbm, uemb_hbm, iemb_hbm,
        logits_hbm, ssq_hbm,
        uidxa_v, uidxb_v, urowsa_v, urowsb_v, didxa_v, didxb_v,
        rowsE0, rowsO0, rowsE1, rowsO1, lout_v, ssq_v,
        sem0, sem1, semu):
    wid = lax.axis_index("c") * NS + lax.axis_index("s")
    base = wid * UPW
    pltpu.sync_copy(it2a_hbm.at[pl.ds(base, UPW)], didxa_v)
    pltpu.sync_copy(it2b_hbm.at[pl.ds(base, UPW)], didxb_v)
    pltpu.sync_copy(u2a_hbm.at[pl.ds(base, UPW)], uidxa_v)
    pltpu.sync_copy(u2b_hbm.at[pl.ds(base, UPW)], uidxb_v)
    pltpu.async_copy(uemb_hbm.at[uidxa_v], urowsa_v, semu)
    pltpu.async_copy(uemb_hbm.at[uidxb_v], urowsb_v, semu)

    zero = jnp.zeros((16,), jnp.float32)
    for rbuf in (rowsE0, rowsO0, rowsE1, rowsO1):
      for r in range(L, LP):
        rbuf[r, pl.ds(0, 16)] = zero

    iota16 = lax.iota(jnp.int32, 16)

    def issue(u, rbufE, rbufO, sem):
      pltpu.async_copy(iemb_hbm.at[didxa_v.at[u]], rbufE.at[pl.ds(0, L)], sem)
      pltpu.async_copy(iemb_hbm.at[didxb_v.at[u]], rbufO.at[pl.ds(0, L)], sem)

    def wait(rbufE, rbufO, sem):
      pltpu.make_async_copy(iemb_hbm.at[didxa_v.at[0]],
                            rbufE.at[pl.ds(0, L)], sem).wait()
      pltpu.make_async_copy(iemb_hbm.at[didxb_v.at[0]],
                            rbufO.at[pl.ds(0, L)], sem).wait()

    rots = [(iota16 + e) & 15 for e in range(16)]
    rows = [iota16 + (c * 16) for c in range(NCHUNK)]

    def compute(u, rbufE, rbufO, ssq):
      u0 = urowsa_v[u, pl.ds(0, 16)]
      u1 = urowsb_v[u, pl.ds(0, 16)]
      ssq = ssq + u0 * u0 + u1 * u1
      uu = jnp.broadcast_to(u, (16,)).astype(jnp.int32)
      accs = [zero] * NCHUNK
      sq = [zero, zero, zero, zero]
      for e in range(16):
        urotA = plsc.load_gather(urowsa_v, [uu, rots[e]])
        urotB = plsc.load_gather(urowsb_v, [uu, rots[e]])
        for c in range(NCHUNK):
          vA = plsc.load_gather(rbufE, [rows[c], rots[e]])
          vB = plsc.load_gather(rbufO, [rows[c], rots[e]])
          accs[c] = accs[c] + (urotA * vA + urotB * vB)
          sq[(e + c) % 4] = sq[(e + c) % 4] + (vA * vA + vB * vB)
      for c in range(NCHUNK):
        lout_v[u, pl.ds(c * 16, 16)] = accs[c]
      return ssq + ((sq[0] + sq[1]) + (sq[2] + sq[3]))

    issue(0, rowsE0, rowsO0, sem0)
    pltpu.make_async_copy(uemb_hbm.at[uidxa_v], urowsa_v, semu).wait()
    pltpu.make_async_copy(uemb_hbm.at[uidxb_v], urowsb_v, semu).wait()

    def pair(i, ssq):
      u0 = 2 * i
      issue(u0 + 1, rowsE1, rowsO1, sem1)
      wait(rowsE0, rowsO0, sem0)
      ssq = compute(u0, rowsE0, rowsO0, ssq)

      @pl.when(i < UPW // 2 - 1)
      def _():
        issue(u0 + 2, rowsE0, rowsO0, sem0)

      wait(rowsE1, rowsO1, sem1)
      ssq = compute(u0 + 1, rowsE1, rowsO1, ssq)
      return ssq

    ssq = lax.fori_loop(0, UPW // 2, pair, zero)
    ssq_v[...] = ssq
    pltpu.sync_copy(lout_v, logits_hbm.at[pl.ds(base, UPW)])
    pltpu.sync_copy(ssq_v, ssq_hbm.at[wid])

  return k(user2a, user2b, items2a, items2b, uemb2, iemb2)


RB = 128
NG = B // RB


def _tc_loss(logits, reward_pad, ssq):
  def body(lg_ref, rw_ref, ssq_ref, gan_ref, reg_ref, acc_ref):
    i = pl.program_id(0)

    @pl.when(i == 0)
    def _():
      acc_ref[0] = 0.0

    lg = lg_ref[...]
    rw = rw_ref[...]
    mask = lax.broadcasted_iota(jnp.int32, (RB, LP), 1) < L
    lgm = jnp.where(mask, lg, -1e30)
    mx = jnp.max(lgm, axis=1, keepdims=True)
    ex = jnp.where(mask, jnp.exp(lgm - mx), 0.0)
    se = jnp.sum(ex, axis=1, keepdims=True)
    lse = mx + jnp.log(se)
    rsum = jnp.sum(rw, axis=1, keepdims=True)
    part = jnp.sum(lg * rw) - jnp.sum(lse * rsum)
    acc_ref[0] = acc_ref[0] + part

    @pl.when(i == NG - 1)
    def _():
      gan_ref[0, 0] = -acc_ref[0] / (B * L)
      reg_ref[0, 0] = REG_SCALE * 0.5 * jnp.sum(ssq_ref[...])

  gan, reg = pl.pallas_call(
      body,
      grid=(NG,),
      in_specs=[
          pl.BlockSpec((RB, LP), lambda i: (i, 0)),
          pl.BlockSpec((RB, LP), lambda i: (i, 0)),
          pl.BlockSpec((NW, 16), lambda i: (0, 0)),
      ],
      out_specs=[
          pl.BlockSpec(memory_space=pltpu.SMEM),
          pl.BlockSpec(memory_space=pltpu.SMEM),
      ],
      out_shape=[
          jax.ShapeDtypeStruct((1, 1), jnp.float32),
          jax.ShapeDtypeStruct((1, 1), jnp.float32),
      ],
      scratch_shapes=[pltpu.SMEM((2,), jnp.float32)],
  )(logits, reward_pad, ssq)
  return gan[0, 0], reg[0, 0]


def kernel(user, items, reward, user_embedding, item_embedding):
  user = user.astype(jnp.int32)
  items = items.astype(jnp.int32)
  user2a = user * 2
  user2b = user2a + 1
  items2a = items * 2
  items2b = items2a + 1
  reward_pad = jnp.pad(reward.astype(jnp.float32), ((0, 0), (0, LP - L)))
  uemb2 = user_embedding.reshape(2 * NROWS, 16)
  iemb2 = item_embedding.reshape(2 * NROWS, 16)
  logits, ssq = _sc_gather_dot(user2a, user2b, items2a, items2b, uemb2, iemb2)
  return _tc_loss(logits, reward_pad, ssq)


# speedup vs baseline: 1.2977x; 1.0351x over previous
import functools

import jax
import jax.numpy as jnp
from jax import lax
from jax.experimental import pallas as pl
from jax.experimental.pallas import tpu as pltpu
from jax.experimental.pallas import tpu_sc as plsc

B = 4096
L = 200
E = 32
LP = 208
NC = 2
NS = 16
NW = NC * NS
UPW = B // NW
NCHUNK = LP // 16
NROWS = 1000000
REG_SCALE = 1e-05


def _sc_gather_dot(user2a, user2b, items2a, items2b, uemb2, iemb2):
  mesh = plsc.VectorSubcoreMesh(core_axis_name="c", subcore_axis_name="s")

  @functools.partial(
      pl.kernel,
      out_type=(
          jax.ShapeDtypeStruct((B, LP), jnp.float32),
          jax.ShapeDtypeStruct((NW, 16), jnp.float32),
      ),
      mesh=mesh,
      scratch_types=[
          pltpu.VMEM((UPW,), jnp.int32),
          pltpu.VMEM((UPW,), jnp.int32),
          pltpu.VMEM((UPW, 16), jnp.float32),
          pltpu.VMEM((UPW, 16), jnp.float32),
          pltpu.VMEM((UPW, L), jnp.int32),
          pltpu.VMEM((UPW, L), jnp.int32),
          pltpu.VMEM((LP, 16), jnp.float32),
          pltpu.VMEM((LP, 16), jnp.float32),
          pltpu.VMEM((LP, 16), jnp.float32),
          pltpu.VMEM((LP, 16), jnp.float32),
          pltpu.VMEM((LP, 16), jnp.float32),
          pltpu.VMEM((LP, 16), jnp.float32),
          pltpu.VMEM((LP, 16), jnp.float32),
          pltpu.VMEM((LP, 16), jnp.float32),
          pltpu.VMEM((LP,), jnp.float32),
          pltpu.VMEM((LP,), jnp.float32),
          pltpu.VMEM((16,), jnp.float32),
          pltpu.SemaphoreType.DMA,
          pltpu.SemaphoreType.DMA,
          pltpu.SemaphoreType.DMA,
          pltpu.SemaphoreType.DMA,
          pltpu.SemaphoreType.DMA,
          pltpu.SemaphoreType.DMA,
      ],
      compiler_params=pltpu.CompilerParams(
          needs_layout_passes=False, use_tc_tiling_on_sc=False),
  )
  def k(u2a_hbm, u2b_hbm, it2a_hbm, it2b_hbm, uemb_hbm, iemb_hbm,
        logits_hbm, ssq_hbm,
        uidxa_v, uidxb_v, urowsa_v, urowsb_v, didxa_v, didxb_v,
        rowsE0, rowsO0, rowsE1, rowsO1, rowsE2, rowsO2, rowsE3, rowsO3,
        lrow0, lrow1, ssq_v, sem0, sem1, sem2, sem3, semu, semL):
    wid = lax.axis_index("c") * NS + lax.axis_index("s")
    base = wid * UPW
    pltpu.sync_copy(it2a_hbm.at[pl.ds(base, UPW)], didxa_v)
    pltpu.sync_copy(it2b_hbm.at[pl.ds(base, UPW)], didxb_v)
    pltpu.sync_copy(u2a_hbm.at[pl.ds(base, UPW)], uidxa_v)
    pltpu.sync_copy(u2b_hbm.at[pl.ds(base, UPW)], uidxb_v)
    pltpu.async_copy(uemb_hbm.at[uidxa_v], urowsa_v, semu)
    pltpu.async_copy(uemb_hbm.at[uidxb_v], urowsb_v, semu)

    zero = jnp.zeros((16,), jnp.float32)
    bufs = ((rowsE0, rowsO0, sem0), (rowsE1, rowsO1, sem1),
            (rowsE2, rowsO2, sem2), (rowsE3, rowsO3, sem3))
    for rbuf in (rowsE0, rowsO0, rowsE1, rowsO1, rowsE2, rowsO2,
                 rowsE3, rowsO3):
      for r in range(L, LP):
        rbuf[r, pl.ds(0, 16)] = zero

    iota16 = lax.iota(jnp.int32, 16)

    def issue(u, rbufE, rbufO, sem):
      pltpu.async_copy(iemb_hbm.at[didxa_v.at[u]], rbufE.at[pl.ds(0, L)], sem)
      pltpu.async_copy(iemb_hbm.at[didxb_v.at[u]], rbufO.at[pl.ds(0, L)], sem)

    def wait(rbufE, rbufO, sem):
      pltpu.make_async_copy(iemb_hbm.at[didxa_v.at[0]],
                            rbufE.at[pl.ds(0, L)], sem).wait()
      pltpu.make_async_copy(iemb_hbm.at[didxb_v.at[0]],
                            rbufO.at[pl.ds(0, L)], sem).wait()

    rots = [(iota16 + e) & 15 for e in range(16)]
    rows = [iota16 + (c * 16) for c in range(NCHUNK)]

    def compute(u, rbufE, rbufO, lrow, ssq):
      u0 = urowsa_v[u, pl.ds(0, 16)]
      u1 = urowsb_v[u, pl.ds(0, 16)]
      ssq = ssq + u0 * u0 + u1 * u1
      uu = jnp.broadcast_to(u, (16,)).astype(jnp.int32)
      accs = [zero] * NCHUNK
      sq = [zero, zero, zero, zero]
      for e in range(16):
        urotA = plsc.load_gather(urowsa_v, [uu, rots[e]])
        urotB = plsc.load_gather(urowsb_v, [uu, rots[e]])
        for c in range(NCHUNK):
          vA = plsc.load_gather(rbufE, [rows[c], rots[e]])
          vB = plsc.load_gather(rbufO, [rows[c], rots[e]])
          accs[c] = accs[c] + (urotA * vA + urotB * vB)
          sq[(e + c) % 4] = sq[(e + c) % 4] + (vA * vA + vB * vB)

      @pl.when(u >= 2)
      def _():
        pltpu.make_async_copy(lrow, logits_hbm.at[base], semL).wait()

      for c in range(NCHUNK):
        lrow[pl.ds(c * 16, 16)] = accs[c]
      pltpu.async_copy(lrow, logits_hbm.at[base + u], semL)
      return ssq + ((sq[0] + sq[1]) + (sq[2] + sq[3]))

    for b in range(4):
      issue(b, bufs[b][0], bufs[b][1], bufs[b][2])
    pltpu.make_async_copy(uemb_hbm.at[uidxa_v], urowsa_v, semu).wait()
    pltpu.make_async_copy(uemb_hbm.at[uidxb_v], urowsb_v, semu).wait()

    def quad(i, ssq):
      u0 = 4 * i
      for b in range(4):
        rbE, rbO, sem = bufs[b]
        wait(rbE, rbO, sem)
        ssq = compute(u0 + b, rbE, rbO, lrow0 if b % 2 == 0 else lrow1, ssq)

        @pl.when(i < UPW // 4 - 1)
        def _():
          issue(u0 + b + 4, rbE, rbO, sem)

      return ssq

    ssq = lax.fori_loop(0, UPW // 4, quad, zero)
    pltpu.make_async_copy(lrow0, logits_hbm.at[base], semL).wait()
    pltpu.make_async_copy(lrow1, logits_hbm.at[base], semL).wait()
    ssq_v[...] = ssq
    pltpu.sync_copy(ssq_v, ssq_hbm.at[wid])

  return k(user2a, user2b, items2a, items2b, uemb2, iemb2)


RB = 128
NG = B // RB


def _tc_loss(logits, reward_pad, ssq):
  def body(lg_ref, rw_ref, ssq_ref, gan_ref, reg_ref, acc_ref):
    i = pl.program_id(0)

    @pl.when(i == 0)
    def _():
      acc_ref[0] = 0.0

    lg = lg_ref[...]
    rw = rw_ref[...]
    mask = lax.broadcasted_iota(jnp.int32, (RB, LP), 1) < L
    lgm = jnp.where(mask, lg, -1e30)
    mx = jnp.max(lgm, axis=1, keepdims=True)
    ex = jnp.where(mask, jnp.exp(lgm - mx), 0.0)
    se = jnp.sum(ex, axis=1, keepdims=True)
    lse = mx + jnp.log(se)
    rsum = jnp.sum(rw, axis=1, keepdims=True)
    part = jnp.sum(lg * rw) - jnp.sum(lse * rsum)
    acc_ref[0] = acc_ref[0] + part

    @pl.when(i == NG - 1)
    def _():
      gan_ref[0, 0] = -acc_ref[0] / (B * L)
      reg_ref[0, 0] = REG_SCALE * 0.5 * jnp.sum(ssq_ref[...])

  gan, reg = pl.pallas_call(
      body,
      grid=(NG,),
      in_specs=[
          pl.BlockSpec((RB, LP), lambda i: (i, 0)),
          pl.BlockSpec((RB, LP), lambda i: (i, 0)),
          pl.BlockSpec((NW, 16), lambda i: (0, 0)),
      ],
      out_specs=[
          pl.BlockSpec(memory_space=pltpu.SMEM),
          pl.BlockSpec(memory_space=pltpu.SMEM),
      ],
      out_shape=[
          jax.ShapeDtypeStruct((1, 1), jnp.float32),
          jax.ShapeDtypeStruct((1, 1), jnp.float32),
      ],
      scratch_shapes=[pltpu.SMEM((2,), jnp.float32)],
  )(logits, reward_pad, ssq)
  return gan[0, 0], reg[0, 0]


def kernel(user, items, reward, user_embedding, item_embedding):
  user = user.astype(jnp.int32)
  items = items.astype(jnp.int32)
  user2a = user * 2
  user2b = user2a + 1
  items2a = items * 2
  items2b = items2a + 1
  reward_pad = jnp.pad(reward.astype(jnp.float32), ((0, 0), (0, LP - L)))
  uemb2 = user_embedding.reshape(2 * NROWS, 16)
  iemb2 = item_embedding.reshape(2 * NROWS, 16)
  logits, ssq = _sc_gather_dot(user2a, user2b, items2a, items2b, uemb2, iemb2)
  return _tc_loss(logits, reward_pad, ssq)
